# baseline (speedup 1.0000x reference)
import functools

import jax
import jax.numpy as jnp
from jax import lax
from jax.experimental import pallas as pl
from jax.experimental.pallas import tpu as pltpu
from jax.experimental.pallas import tpu_sc as plsc

NC = 2
NS = 16
NW = NC * NS

CHUNK = 128


@functools.partial(jax.jit, static_argnums=(2, 3))
def _sc_gather(ids_flat, table, total, d):
    rows_per_w = total // NW
    nchunk = rows_per_w // CHUNK

    mesh = plsc.VectorSubcoreMesh(core_axis_name="c", subcore_axis_name="s")

    @functools.partial(
        pl.kernel,
        out_type=jax.ShapeDtypeStruct((total, d), jnp.float32),
        mesh=mesh,
        scratch_types=[
            pltpu.VMEM((rows_per_w,), jnp.int32),
            pltpu.VMEM((CHUNK, d), jnp.float32),
            pltpu.SemaphoreType.DMA,
        ],
        compiler_params=pltpu.CompilerParams(use_tc_tiling_on_sc=False),
    )
    def body(ids_hbm, table_hbm, out_hbm, idx_v, rows_v, sem):
        wid = lax.axis_index("s") * NC + lax.axis_index("c")
        base = wid * rows_per_w
        pltpu.sync_copy(ids_hbm.at[pl.ds(base, rows_per_w)], idx_v)

        @pl.loop(0, nchunk)
        def _chunk(j):
            off = j * CHUNK
            pltpu.async_copy(
                table_hbm.at[idx_v.at[pl.ds(off, CHUNK)]], rows_v, sem
            ).wait()
            pltpu.sync_copy(rows_v, out_hbm.at[pl.ds(base + off, CHUNK)])

    return body(ids_flat, table)


def kernel(ids, embd):
    b, f = ids.shape
    v, d = embd.shape
    total = b * f
    ids_flat = ids.reshape(total).astype(jnp.int32)
    out = _sc_gather(ids_flat, embd, total, d)
    return out.reshape(b, f, d)


# speedup vs baseline: 1.0806x; 1.0806x over previous
import functools

import jax
import jax.numpy as jnp
from jax import lax
from jax.experimental import pallas as pl
from jax.experimental.pallas import tpu as pltpu
from jax.experimental.pallas import tpu_sc as plsc

NC = 2
NS = 16
NW = NC * NS

CHUNK = 256
NBUF = 4
K_AHEAD = 2


@functools.partial(jax.jit, static_argnums=(2, 3))
def _sc_gather(ids_flat, table, total, d):
    rows_per_w = total // NW
    nchunk = rows_per_w // CHUNK
    assert nchunk % NBUF == 0 and nchunk >= 2 * NBUF

    mesh = plsc.VectorSubcoreMesh(core_axis_name="c", subcore_axis_name="s")

    @functools.partial(
        pl.kernel,
        out_type=jax.ShapeDtypeStruct((total, d), jnp.float32),
        mesh=mesh,
        scratch_types=(
            [pltpu.VMEM((rows_per_w,), jnp.int32)]
            + [pltpu.VMEM((CHUNK, d), jnp.float32) for _ in range(NBUF)]
            + [pltpu.SemaphoreType.DMA for _ in range(2 * NBUF)]
        ),
        compiler_params=pltpu.CompilerParams(use_tc_tiling_on_sc=False),
    )
    def body(ids_hbm, table_hbm, out_hbm, idx_v, *rest):
        bufs = rest[:NBUF]
        gsems = rest[NBUF:2 * NBUF]
        wsems = rest[2 * NBUF:]

        wid = lax.axis_index("s") * NC + lax.axis_index("c")
        base = wid * rows_per_w
        pltpu.sync_copy(ids_hbm.at[pl.ds(base, rows_per_w)], idx_v)

        def start_gather(c, b):
            pltpu.async_copy(
                table_hbm.at[idx_v.at[pl.ds(c * CHUNK, CHUNK)]],
                bufs[b], gsems[b])

        def wait_gather(b):
            pltpu.make_async_copy(
                table_hbm.at[idx_v.at[pl.ds(0, CHUNK)]],
                bufs[b], gsems[b]).wait()

        def start_writeback(c, b):
            pltpu.async_copy(
                bufs[b], out_hbm.at[pl.ds(base + c * CHUNK, CHUNK)],
                wsems[b])

        def wait_writeback(b):
            pltpu.make_async_copy(
                bufs[b], out_hbm.at[pl.ds(base, CHUNK)], wsems[b]).wait()

        for c in range(K_AHEAD):
            start_gather(c, c)

        @pl.loop(0, nchunk // NBUF)
        def _outer(o):
            for b in range(NBUF):
                c = o * NBUF + b
                nb = (b + K_AHEAD) % NBUF
                ahead = c + K_AHEAD

                @pl.when(jnp.logical_and(ahead >= NBUF, ahead < nchunk))
                def _steady():
                    wait_writeback(nb)
                    start_gather(ahead, nb)

                @pl.when(ahead < NBUF)
                def _warmup():
                    start_gather(ahead, nb)

                wait_gather(b)
                start_writeback(c, b)

        for b in range(NBUF):
            wait_writeback(b)

    return body(ids_flat, table)


def kernel(ids, embd):
    b, f = ids.shape
    v, d = embd.shape
    total = b * f
    ids_flat = ids.reshape(total).astype(jnp.int32)
    out = _sc_gather(ids_flat, embd, total, d)
    return out.reshape(b, f, d)
